# baseline (speedup 1.0000x reference)
import functools

import jax
import jax.numpy as jnp
from jax import lax
from jax.experimental import pallas as pl
from jax.experimental.pallas import tpu as pltpu
from jax.experimental.pallas import tpu_sc as plsc

N = 10000
E = 320000
DF = 128
DE = 16
DT = 16
DA = DE + DT
DO = 128

NC = 2
NS = 16
NPAD = 10240
ROWS_PER_TILE = NPAD // NS
EDGES_PER_CORE = E // NC
EDGES_PER_TILE = EDGES_PER_CORE // NS
CHUNK = 40
NCHUNKS = EDGES_PER_TILE // CHUNK
NSLOTS = 4


def _sc_aggregate(ei, x, edge_attr, edge_t):
  mesh = plsc.VectorSubcoreMesh(core_axis_name="c", subcore_axis_name="s")

  slot_scratch = []
  for _ in range(NSLOTS):
    slot_scratch += [
        pltpu.VMEM((2, CHUNK), jnp.int32),
        pltpu.VMEM((CHUNK, DF), jnp.float32),
        pltpu.VMEM((CHUNK, DA), jnp.float32),
        pltpu.SemaphoreType.DMA,
        pltpu.SemaphoreType.DMA,
        pltpu.SemaphoreType.DMA,
    ]

  @functools.partial(
      pl.kernel,
      out_type=(
          jax.ShapeDtypeStruct((NC, NPAD, DF), jnp.float32),
          jax.ShapeDtypeStruct((NC, NPAD, DA), jnp.float32),
      ),
      mesh=mesh,
      compiler_params=pltpu.CompilerParams(use_tc_tiling_on_sc=False),
      scratch_types=slot_scratch + [
          pltpu.VMEM_SHARED((NPAD, DF), jnp.float32),
          pltpu.VMEM_SHARED((NPAD, DA), jnp.float32),
      ],
  )
  def agg_kernel(ei_h, x_h, ea_h, et_h, px_h, pat_h, *sc):
    idxs = [sc[6 * k + 0] for k in range(NSLOTS)]
    xrs = [sc[6 * k + 1] for k in range(NSLOTS)]
    eats = [sc[6 * k + 2] for k in range(NSLOTS)]
    sis = [sc[6 * k + 3] for k in range(NSLOTS)]
    sgs = [sc[6 * k + 4] for k in range(NSLOTS)]
    sss = [sc[6 * k + 5] for k in range(NSLOTS)]
    aggx = sc[6 * NSLOTS]
    aggat = sc[6 * NSLOTS + 1]

    c = lax.axis_index("c")
    s = lax.axis_index("s")
    rbase = s * ROWS_PER_TILE

    zeros16 = jnp.zeros((16,), jnp.float32)

    @pl.loop(0, CHUNK)
    def _(i):
      @pl.loop(0, DA // 16)
      def _(k):
        eats[0][i, pl.ds(k * 16, 16)] = zeros16

      @pl.loop(0, DF // 16)
      def _(k):
        xrs[0][i, pl.ds(k * 16, 16)] = zeros16

    @pl.loop(0, ROWS_PER_TILE // CHUNK)
    def _(k):
      dst = rbase + k * CHUNK
      pltpu.sync_copy(xrs[0], aggx.at[pl.ds(dst, CHUNK)])
      pltpu.sync_copy(eats[0], aggat.at[pl.ds(dst, CHUNK)])

    plsc.subcore_barrier()

    ebase = c * EDGES_PER_CORE + s * EDGES_PER_TILE
    last = NCHUNKS - 1

    def idx_start(j, b):
      off = ebase + jnp.minimum(j, last) * CHUNK
      pltpu.async_copy(ei_h.at[:, pl.ds(off, CHUNK)], idxs[b], sis[b])
      pltpu.async_copy(ea_h.at[pl.ds(off, CHUNK)], eats[b].at[:, pl.ds(0, DE)],
                       sis[b])
      pltpu.async_copy(et_h.at[pl.ds(off, CHUNK)], eats[b].at[:, pl.ds(DE, DT)],
                       sis[b])

    def idx_wait(b):
      pltpu.make_async_copy(ei_h.at[:, pl.ds(0, CHUNK)], idxs[b], sis[b]).wait()
      pltpu.make_async_copy(ea_h.at[pl.ds(0, CHUNK)],
                            eats[b].at[:, pl.ds(0, DE)], sis[b]).wait()
      pltpu.make_async_copy(et_h.at[pl.ds(0, CHUNK)],
                            eats[b].at[:, pl.ds(DE, DT)], sis[b]).wait()

    def gather_start(b):
      pltpu.async_copy(x_h.at[idxs[b].at[1]], xrs[b], sgs[b])

    def gather_wait(b):
      pltpu.make_async_copy(x_h.at[idxs[b].at[1]], xrs[b], sgs[b]).wait()

    def scat_start(b):
      pltpu.async_copy(xrs[b], aggx.at[idxs[b].at[0]], sss[b], add=True)
      pltpu.async_copy(eats[b], aggat.at[idxs[b].at[0]], sss[b], add=True)

    def scat_wait(b):
      pltpu.make_async_copy(xrs[b], aggx.at[idxs[b].at[0]], sss[b]).wait()
      pltpu.make_async_copy(eats[b], aggat.at[idxs[b].at[0]], sss[b]).wait()

    for b in range(NSLOTS):
      idx_start(b, b)

    @pl.loop(0, NCHUNKS // NSLOTS)
    def _(i):
      j0 = NSLOTS * i
      idx_wait(0)
      gather_start(0)
      idx_wait(1)
      gather_start(1)
      gather_wait(0)
      scat_start(0)
      idx_wait(2)
      gather_start(2)
      gather_wait(1)
      scat_start(1)
      idx_wait(3)
      gather_start(3)
      scat_wait(0)
      idx_start(j0 + 4, 0)
      gather_wait(2)
      scat_start(2)
      scat_wait(1)
      idx_start(j0 + 5, 1)
      gather_wait(3)
      scat_start(3)
      scat_wait(2)
      idx_start(j0 + 6, 2)
      scat_wait(3)
      idx_start(j0 + 7, 3)

    for b in range(NSLOTS):
      idx_wait(b)
      if b < NCHUNKS % NSLOTS:
        gather_start(b)
        gather_wait(b)
        scat_start(b)
        scat_wait(b)

    plsc.subcore_barrier()
    pltpu.sync_copy(aggx.at[pl.ds(rbase, ROWS_PER_TILE)],
                    px_h.at[c, pl.ds(rbase, ROWS_PER_TILE)])
    pltpu.sync_copy(aggat.at[pl.ds(rbase, ROWS_PER_TILE)],
                    pat_h.at[c, pl.ds(rbase, ROWS_PER_TILE)])

  return agg_kernel(ei, x, edge_attr, edge_t)


BLK = 1000


def _tc_combine(px, pat, x, wmx, wmat, wr, bm, br):
  def body(px_r, pat_r, x_r, wmx_r, wmat_r, wr_r, bm_r, br_r, o_r):
    aggx = px_r[0] + px_r[1]
    aggat = pat_r[0] + pat_r[1]
    acc = jnp.dot(aggx, wmx_r[...], preferred_element_type=jnp.float32)
    acc = acc + jnp.dot(aggat, wmat_r[...], preferred_element_type=jnp.float32)
    acc = 0.5 * (acc + bm_r[...])
    acc = acc + jnp.dot(x_r[...], wr_r[...], preferred_element_type=jnp.float32)
    o_r[...] = acc + br_r[...]

  return pl.pallas_call(
      body,
      grid=(N // BLK,),
      in_specs=[
          pl.BlockSpec((NC, BLK, DF), lambda i: (0, i, 0)),
          pl.BlockSpec((NC, BLK, DA), lambda i: (0, i, 0)),
          pl.BlockSpec((BLK, DF), lambda i: (i, 0)),
          pl.BlockSpec((DF, DO), lambda i: (0, 0)),
          pl.BlockSpec((DA, DO), lambda i: (0, 0)),
          pl.BlockSpec((DF, DO), lambda i: (0, 0)),
          pl.BlockSpec((1, DO), lambda i: (0, 0)),
          pl.BlockSpec((1, DO), lambda i: (0, 0)),
      ],
      out_specs=pl.BlockSpec((BLK, DO), lambda i: (i, 0)),
      out_shape=jax.ShapeDtypeStruct((N, DO), jnp.float32),
  )(px, pat, x, wmx, wmat, wr, bm, br)


def kernel(x, edge_index, edge_attr, edge_t, W_m, b_m, W_r, b_r):
  ei = edge_index.astype(jnp.int32)
  px, pat = _sc_aggregate(ei, x, edge_attr, edge_t)
  wmx = W_m[:DF]
  wmat = W_m[DF:]
  bm = b_m.reshape(1, DO)
  br = b_r.reshape(1, DO)
  return _tc_combine(px, pat, x, wmx, wmat, W_r, bm, br)
